# Initial kernel scaffold; baseline (speedup 1.0000x reference)
import jax
import jax.numpy as jnp
from jax.experimental import pallas as pl


def kernel(x, edge_index_x, ptr_x, y, edge_index_y, ptr_y, emb, w_skip, b_skip, w0, b0, w1, b1, g0, be0, g1, be1):
    raise NotImplementedError("write your pallas kernel here")


# speedup vs baseline: 10.3644x; 10.3644x over previous
import functools

import jax
import jax.numpy as jnp
from jax import lax
from jax.experimental import pallas as pl
from jax.experimental.pallas import tpu as pltpu
from jax.experimental.pallas import tpu_sc as plsc

NC = 2
NS = 16
CH = 128
ZR = 64
DW = 128


def _ceil_to(a, m):
    return (a + m - 1) // m * m


def _prep_body(nrows_x, erows_per_tile, erows_per_core,
               x2d, dst2d, emb, z16, o16,
               embx_out, deg_out,
               idx_v, gbuf, z16_v, o16_v, sem, deg_sh):
    c = lax.axis_index("c")
    s = lax.axis_index("s")
    wid = s * NC + c
    nw = NC * NS
    pltpu.sync_copy(z16, z16_v)
    pltpu.sync_copy(o16, o16_v)

    def gather_row(row):
        pltpu.sync_copy(x2d.at[row], idx_v)
        pltpu.async_copy(emb.at[idx_v], gbuf, sem).wait()
        pltpu.sync_copy(gbuf, embx_out.at[pl.ds(row * CH, CH)])

    lo = nrows_x // nw
    rem = nrows_x % nw
    for r in range(lo):
        gather_row(wid * lo + r)
    if rem:
        @pl.when(wid < rem)
        def _():
            gather_row(nw * lo + wid)

    npad = deg_sh.shape[0]
    base = s * (npad // NS)
    for k in range(npad // NS // ZR):
        pltpu.sync_copy(z16_v, deg_sh.at[pl.ds(base + k * ZR, ZR)])
    plsc.subcore_barrier()

    erow0 = c * erows_per_core + s * erows_per_tile

    @pl.loop(0, erows_per_tile)
    def _(i):
        pltpu.sync_copy(dst2d.at[erow0 + i], idx_v)
        pltpu.sync_copy(o16_v, deg_sh.at[idx_v], add=True)

    plsc.subcore_barrier()
    for k in range(npad // NS // ZR):
        pltpu.sync_copy(deg_sh.at[pl.ds(base + k * ZR, ZR)], z16_v)
        pltpu.sync_copy(z16_v, deg_out.at[c, pl.ds(base + k * ZR, ZR)])


def _edge_body(erows_per_tile, erows_per_core,
               src2d, dst2d, hv, z64,
               acc_out,
               sidx, didx, rows_v, zb, sem, acc_sh):
    c = lax.axis_index("c")
    s = lax.axis_index("s")
    npad = acc_sh.shape[0]
    base = s * (npad // NS)
    pltpu.sync_copy(z64, zb)
    for k in range(npad // NS // ZR):
        pltpu.sync_copy(zb, acc_sh.at[pl.ds(base + k * ZR, ZR)])
    plsc.subcore_barrier()

    erow0 = c * erows_per_core + s * erows_per_tile

    @pl.loop(0, erows_per_tile)
    def _(i):
        pltpu.sync_copy(src2d.at[erow0 + i], sidx)
        pltpu.sync_copy(dst2d.at[erow0 + i], didx)
        pltpu.async_copy(hv.at[sidx], rows_v, sem).wait()
        pltpu.sync_copy(rows_v, acc_sh.at[didx], add=True)

    plsc.subcore_barrier()
    for k in range(npad // NS // ZR):
        pltpu.sync_copy(acc_sh.at[pl.ds(base + k * ZR, ZR)], zb)
        pltpu.sync_copy(zb, acc_out.at[c, pl.ds(base + k * ZR, ZR)])


def _dinv_from(degp_ref):
    deg = degp_ref[0, :, 0:1] + degp_ref[1, :, 0:1] + 1.0
    return lax.rsqrt(deg)


def _tc1_body(embx_ref, w0_ref, degp_ref, hv0_ref):
    dinv = _dinv_from(degp_ref)
    hw = jnp.dot(embx_ref[...], w0_ref[...], preferred_element_type=jnp.float32)
    hv0_ref[...] = dinv * hw


def _ln_relu(t, g_ref, be_ref):
    mu = jnp.mean(t, axis=-1, keepdims=True)
    ctr = t - mu
    var = jnp.mean(ctr * ctr, axis=-1, keepdims=True)
    return jnp.maximum(ctr * lax.rsqrt(var + 1e-5) * g_ref[...] + be_ref[...], 0.0)


def _tc2_body(acc_ref, hv0_ref, degp_ref, embx_ref, wskip_ref, bskip_ref,
              w1_ref, b0_ref, g0_ref, be0_ref, hv1_ref):
    dinv = _dinv_from(degp_ref)
    t = dinv * (acc_ref[0] + acc_ref[1] + hv0_ref[...]) + b0_ref[...]
    h = _ln_relu(t, g0_ref, be0_ref)
    u = jnp.dot(embx_ref[...], wskip_ref[...],
                preferred_element_type=jnp.float32) + bskip_ref[...] + h
    hv1_ref[...] = dinv * jnp.dot(u, w1_ref[...],
                                  preferred_element_type=jnp.float32)


def _tc3_body(acc_ref, hv1_ref, degp_ref, b1_ref, g1_ref, be1_ref,
              lo_ref, hi_ref, ci_ref, h2_ref, g_ref):
    i = pl.program_id(0)
    rb = hv1_ref.shape[0]
    nb = lo_ref.shape[0]
    dinv = _dinv_from(degp_ref)
    t = dinv * (acc_ref[0] + acc_ref[1] + hv1_ref[...]) + b1_ref[...]
    h2 = _ln_relu(t, g1_ref, be1_ref)
    h2_ref[...] = h2
    gi = (lax.broadcasted_iota(jnp.int32, (nb, rb), 1) + i * rb).astype(jnp.float32)
    seg = jnp.where((gi >= lo_ref[...]) & (gi < hi_ref[...]), 1.0, 0.0)

    @pl.when(i == 0)
    def _():
        g_ref[...] = jnp.zeros_like(g_ref)

    g_ref[...] += jnp.dot(seg, h2, preferred_element_type=jnp.float32)

    @pl.when(i == pl.num_programs(0) - 1)
    def _():
        g_ref[...] = g_ref[...] * ci_ref[...]


def kernel(x, edge_index_x, ptr_x, y, edge_index_y, ptr_y, emb, w_skip,
           b_skip, w0, b0, w1, b1, g0, be0, g1, be1):
    n = x.shape[0]
    e = edge_index_x.shape[1]
    d = emb.shape[1]
    nb = ptr_x.shape[0] - 1
    f32 = jnp.float32

    npad = _ceil_to(n, NS * ZR)
    nrows_x = npad // CH
    erows = _ceil_to((e + CH - 1) // CH, NC * NS)
    erows_per_tile = erows // (NC * NS)
    erows_per_core = erows // NC
    epad = erows * CH

    x_pad = jnp.concatenate(
        [x.astype(jnp.int32), jnp.zeros((npad - n,), jnp.int32)]).reshape(nrows_x, CH)
    src = edge_index_x[0].astype(jnp.int32)
    dst = edge_index_x[1].astype(jnp.int32)
    src2d = jnp.concatenate(
        [src, jnp.zeros((epad - e,), jnp.int32)]).reshape(erows, CH)
    dst2d = jnp.concatenate(
        [dst, jnp.full((epad - e,), n, jnp.int32)]).reshape(erows, CH)
    z16 = jnp.zeros((ZR, DW), f32)
    o16 = jnp.ones((CH, DW), f32)
    z64 = jnp.zeros((ZR, d), f32)

    mesh = plsc.VectorSubcoreMesh(core_axis_name="c", subcore_axis_name="s")

    prep = pl.kernel(
        functools.partial(_prep_body, nrows_x, erows_per_tile, erows_per_core),
        out_type=(
            jax.ShapeDtypeStruct((npad, d), f32),
            jax.ShapeDtypeStruct((NC, npad, DW), f32),
        ),
        mesh=mesh,
        scratch_types=[
            pltpu.VMEM((CH,), jnp.int32),
            pltpu.VMEM((CH, d), f32),
            pltpu.VMEM((ZR, DW), f32),
            pltpu.VMEM((CH, DW), f32),
            pltpu.SemaphoreType.DMA,
            pltpu.VMEM_SHARED((npad, DW), f32),
        ],
    )

    edge_pass = pl.kernel(
        functools.partial(_edge_body, erows_per_tile, erows_per_core),
        out_type=jax.ShapeDtypeStruct((NC, npad, d), f32),
        mesh=mesh,
        scratch_types=[
            pltpu.VMEM((CH,), jnp.int32),
            pltpu.VMEM((CH,), jnp.int32),
            pltpu.VMEM((CH, d), f32),
            pltpu.VMEM((ZR, d), f32),
            pltpu.SemaphoreType.DMA,
            pltpu.VMEM_SHARED((npad, d), f32),
        ],
    )

    embx, degp = prep(x_pad, dst2d, emb, z16, o16)

    rb = 1024
    grid = npad // rb
    row_spec = pl.BlockSpec((rb, d), lambda i: (i, 0))
    acc_spec = pl.BlockSpec((NC, rb, d), lambda i: (0, i, 0))
    deg_spec = pl.BlockSpec((NC, rb, DW), lambda i: (0, i, 0))
    mat_spec = pl.BlockSpec((d, d), lambda i: (0, 0))
    vec_spec = pl.BlockSpec((1, d), lambda i: (0, 0))

    hv0 = pl.pallas_call(
        _tc1_body,
        grid=(grid,),
        in_specs=[row_spec, mat_spec, deg_spec],
        out_specs=row_spec,
        out_shape=jax.ShapeDtypeStruct((npad, d), f32),
    )(embx, w0, degp)

    acc0 = edge_pass(src2d, dst2d, hv0, z64)

    b0r = b0.reshape(1, d)
    g0r = g0.reshape(1, d)
    be0r = be0.reshape(1, d)
    bskipr = b_skip.reshape(1, d)
    hv1 = pl.pallas_call(
        _tc2_body,
        grid=(grid,),
        in_specs=[acc_spec, row_spec, deg_spec, row_spec, mat_spec, vec_spec,
                  mat_spec, vec_spec, vec_spec, vec_spec],
        out_specs=row_spec,
        out_shape=jax.ShapeDtypeStruct((npad, d), f32),
    )(acc0, hv0, degp, embx, w_skip, bskipr, w1, b0r, g0r, be0r)

    acc1 = edge_pass(src2d, dst2d, hv1, z64)

    lo = ptr_x[:-1].astype(f32).reshape(nb, 1)
    hi = ptr_x[1:].astype(f32).reshape(nb, 1)
    ci = 1.0 / jnp.maximum(hi - lo, 1.0)
    b1r = b1.reshape(1, d)
    g1r = g1.reshape(1, d)
    be1r = be1.reshape(1, d)
    seg_spec = pl.BlockSpec((nb, 1), lambda i: (0, 0))
    h2_pad, g_x = pl.pallas_call(
        _tc3_body,
        grid=(grid,),
        in_specs=[acc_spec, row_spec, deg_spec, vec_spec, vec_spec, vec_spec,
                  seg_spec, seg_spec, seg_spec],
        out_specs=[row_spec, pl.BlockSpec((nb, d), lambda i: (0, 0))],
        out_shape=[jax.ShapeDtypeStruct((npad, d), f32),
                   jax.ShapeDtypeStruct((nb, d), f32)],
    )(acc1, hv1, degp, b1r, g1r, be1r, lo, hi, ci)

    return (h2_pad[:n], g_x)
